# Initial kernel scaffold; baseline (speedup 1.0000x reference)
import jax
import jax.numpy as jnp
from jax.experimental import pallas as pl


def kernel(feature, edges, weight, poi_emb, cat_emb, lat_emb, long_emb, W_in, b_in, gat_W, att_src, att_dst, gat_b, W_og, b_og, fc1_W, fc1_b, fc2_W, fc2_b):
    raise NotImplementedError("write your pallas kernel here")


# speedup vs baseline: 52.5864x; 52.5864x over previous
import functools

import jax
import jax.numpy as jnp
from jax import lax
from jax.experimental import pallas as pl
from jax.experimental.pallas import tpu as pltpu

NPAD = 768
F32 = jnp.float32


def _leaky(x, slope):
    return jnp.where(x >= 0, x, slope * x)


def _proj_body(t_ref, w_ref, o_ref):
    o_ref[...] = jnp.dot(t_ref[...], w_ref[...], preferred_element_type=F32)


def _project_table(table, w):
    v, k = table.shape
    bm = 512
    grid = (pl.cdiv(v, bm),)
    return pl.pallas_call(
        _proj_body,
        grid=grid,
        in_specs=[
            pl.BlockSpec((bm, k), lambda i: (i, 0)),
            pl.BlockSpec((k, 128), lambda i: (0, 0)),
        ],
        out_specs=pl.BlockSpec((bm, 128), lambda i: (i, 0)),
        out_shape=jax.ShapeDtypeStruct((v, 128), F32),
    )(table, w)


def _mega_body(n_real, c_ref, xw_ref, gatw_ref, asrc_ref, adst_ref, gatb_ref,
               wog_ref, bog_ref, fc1_ref, fc1b_ref, fc2_ref, fc2b_ref,
               bin_ref, o_ref):
    c = c_ref[0]
    row = lax.broadcasted_iota(jnp.int32, (NPAD, NPAD), 0)
    col = lax.broadcasted_iota(jnp.int32, (NPAD, NPAD), 1)
    cf = c + jnp.where((row == col) & (row < n_real), 1.0, 0.0)
    mask = cf > 0
    deg = jnp.sum(cf, axis=1, keepdims=True)
    dinv = jnp.where(deg > 0, lax.rsqrt(deg), 0.0)

    rmask = lax.broadcasted_iota(jnp.int32, (NPAD, 1), 0) < n_real
    xw = jnp.where(rmask, xw_ref[0], 0.0)

    h = jnp.dot(cf, dinv * xw, preferred_element_type=F32) * dinv + bin_ref[...]
    h = _leaky(h, 0.01)

    for i in range(3):
        wg = gatw_ref[i]
        a_s = asrc_ref[i]
        a_d = adst_ref[i]
        bg = gatb_ref[i]

        def gat(hh, wg=wg, a_s=a_s, a_d=a_d, bg=bg):
            xwg = jnp.dot(hh, wg, preferred_element_type=F32)
            al_s = lax.dot_general(a_s, xwg, (((1,), (1,)), ((), ())),
                                   preferred_element_type=F32)
            al_d = lax.dot_general(xwg, a_d, (((1,), (1,)), ((), ())),
                                   preferred_element_type=F32)
            e = _leaky(al_d + al_s, 0.2)
            m = jnp.max(jnp.where(mask, e, -1e30), axis=1, keepdims=True)
            ex = jnp.where(mask, jnp.exp(e - m) * cf, 0.0)
            den = jnp.sum(ex, axis=1, keepdims=True)
            alpha = ex * (1.0 / (den + 1e-16))
            return jnp.dot(alpha, xwg, preferred_element_type=F32) + bg

        t = gat(h)
        h2 = _leaky(t, 0.01) + t
        h = _leaky(gat(h2), 0.01)

    hw = jnp.dot(h, wog_ref[...], preferred_element_type=F32)
    og = jnp.dot(cf, dinv * hw, preferred_element_type=F32) * dinv + bog_ref[...]
    og = _leaky(og, 0.01)

    h1 = jnp.sum(fc1_ref[...] * og, axis=0, keepdims=True) + fc1b_ref[...]
    h1 = _leaky(h1, 0.01)
    o_ref[0] = jnp.dot(h1, fc2_ref[...], preferred_element_type=F32) + fc2b_ref[...]


def _run_graph_stack(b, n_real, c, xw, gat_w, att_src, att_dst, gat_b,
                     w_og, b_og, fc1_w, fc1_b, fc2_w, fc2_b, b_in):
    fixed = lambda *z: tuple(0 for _ in z)
    out = pl.pallas_call(
        functools.partial(_mega_body, n_real),
        grid=(b,),
        in_specs=[
            pl.BlockSpec((1, NPAD, NPAD), lambda g: (g, 0, 0)),
            pl.BlockSpec((1, NPAD, 128), lambda g: (g, 0, 0)),
            pl.BlockSpec((3, 128, 128), lambda g: (0, 0, 0)),
            pl.BlockSpec((3, 1, 128), lambda g: (0, 0, 0)),
            pl.BlockSpec((3, 1, 128), lambda g: (0, 0, 0)),
            pl.BlockSpec((3, 1, 128), lambda g: (0, 0, 0)),
            pl.BlockSpec((128, 1), lambda g: (0, 0)),
            pl.BlockSpec((1, 1), lambda g: (0, 0)),
            pl.BlockSpec((NPAD, 128), lambda g: (0, 0)),
            pl.BlockSpec((1, 128), lambda g: (0, 0)),
            pl.BlockSpec((128, 128), lambda g: (0, 0)),
            pl.BlockSpec((1, 128), lambda g: (0, 0)),
            pl.BlockSpec((1, 128), lambda g: (0, 0)),
        ],
        out_specs=pl.BlockSpec((1, 1, 128), lambda g: (g, 0, 0)),
        out_shape=jax.ShapeDtypeStruct((b, 1, 128), F32),
        compiler_params=pltpu.CompilerParams(
            dimension_semantics=("arbitrary",),
        ),
    )(c, xw, gat_w, att_src, att_dst, gat_b, w_og, b_og, fc1_w, fc1_b,
      fc2_w, fc2_b, b_in)
    return out.reshape(b, 128)


def kernel(feature, edges, weight, poi_emb, cat_emb, lat_emb, long_emb, W_in,
           b_in, gat_W, att_src, att_dst, gat_b, W_og, b_og, fc1_W, fc1_b,
           fc2_W, fc2_b):
    b, n, _ = feature.shape
    e = edges.shape[2]
    nn = b * n

    tpoi = _project_table(poi_emb, W_in[0:300])
    tcat = _project_table(cat_emb, W_in[300:400])
    tlat = _project_table(lat_emb, W_in[400:600])
    tlon = _project_table(long_emb, W_in[600:800])

    f = feature.reshape(nn, 5)
    xw = tpoi[f[:, 0]] + tcat[f[:, 1]] + tlat[f[:, 3]] + tlon[f[:, 4]]
    xw = jnp.pad(xw.reshape(b, n, 128), ((0, 0), (0, NPAD - n), (0, 0)))

    src = edges[:, 0, :].astype(jnp.int32)
    dst = edges[:, 1, :].astype(jnp.int32)
    flat = (jnp.arange(b, dtype=jnp.int32)[:, None] * (NPAD * NPAD)
            + dst * NPAD + src).reshape(-1)
    c = jax.ops.segment_sum(jnp.ones((b * e,), F32), flat,
                            num_segments=b * NPAD * NPAD)
    c = c.reshape(b, NPAD, NPAD)

    return _run_graph_stack(
        b, n, c, xw, gat_W,
        att_src.reshape(3, 1, 128), att_dst.reshape(3, 1, 128),
        gat_b.reshape(3, 1, 128), W_og, b_og.reshape(1, 1),
        jnp.pad(fc1_W, ((0, NPAD - n), (0, 0))), fc1_b.reshape(1, 128),
        fc2_W, fc2_b.reshape(1, 128), b_in.reshape(1, 128))
